# Initial kernel scaffold; baseline (speedup 1.0000x reference)
import jax
import jax.numpy as jnp
from jax.experimental import pallas as pl


def kernel(x, table):
    raise NotImplementedError("write your pallas kernel here")


# speedup vs baseline: 1.3702x; 1.3702x over previous
import jax
import jax.numpy as jnp
from jax.experimental import pallas as pl


_BL = 1024


def _add_kernel(x_ref, t_ref, o_ref):
    o_ref[...] = x_ref[...] + t_ref[...]


def kernel(x, table):
    B, L, D = x.shape
    grid = (B, L // _BL)
    return pl.pallas_call(
        _add_kernel,
        grid=grid,
        in_specs=[
            pl.BlockSpec((1, _BL, D), lambda b, i: (b, i, 0)),
            pl.BlockSpec((_BL, D), lambda b, i: (i, 0)),
        ],
        out_specs=pl.BlockSpec((1, _BL, D), lambda b, i: (b, i, 0)),
        out_shape=jax.ShapeDtypeStruct((B, L, D), x.dtype),
    )(x, table)


# speedup vs baseline: 1.6808x; 1.2267x over previous
import jax
import jax.numpy as jnp
from jax.experimental import pallas as pl


_BL = 1024


def _add_kernel(x_ref, t_ref, o_ref):
    o_ref[...] = x_ref[...] + t_ref[...]


def kernel(x, table):
    B, L, D = x.shape
    grid = (L // _BL, B)
    return pl.pallas_call(
        _add_kernel,
        grid=grid,
        in_specs=[
            pl.BlockSpec((1, _BL, D), lambda i, b: (b, i, 0)),
            pl.BlockSpec((_BL, D), lambda i, b: (i, 0)),
        ],
        out_specs=pl.BlockSpec((1, _BL, D), lambda i, b: (b, i, 0)),
        out_shape=jax.ShapeDtypeStruct((B, L, D), x.dtype),
    )(x, table)


# speedup vs baseline: 1.8007x; 1.0713x over previous
import jax
import jax.numpy as jnp
from jax.experimental import pallas as pl


_BL = 2048


def _add_kernel(x_ref, t_ref, o_ref):
    o_ref[...] = x_ref[...] + t_ref[...]


def kernel(x, table):
    B, L, D = x.shape
    grid = (L // _BL, B)
    return pl.pallas_call(
        _add_kernel,
        grid=grid,
        in_specs=[
            pl.BlockSpec((1, _BL, D), lambda i, b: (b, i, 0)),
            pl.BlockSpec((_BL, D), lambda i, b: (i, 0)),
        ],
        out_specs=pl.BlockSpec((1, _BL, D), lambda i, b: (b, i, 0)),
        out_shape=jax.ShapeDtypeStruct((B, L, D), x.dtype),
    )(x, table)
